# baseline (speedup 1.0000x reference)
import functools

import jax
import jax.numpy as jnp
from jax import lax
from jax.experimental import pallas as pl
from jax.experimental.pallas import tpu as pltpu
from jax.experimental.pallas import tpu_sc as plsc

BB = 8
NC, NS = 2, 16
L = 16


def _rank_body(rand_ref, rank_ref):
    r = rand_ref[...]
    bb, c = r.shape
    rj = r[:, :, None]
    rk = r[:, None, :]
    jlt = jax.lax.broadcasted_iota(jnp.int32, (bb, c, c), 1) < \
        jax.lax.broadcasted_iota(jnp.int32, (bb, c, c), 2)
    lt_f = jnp.where(rj < rk, 1.0, 0.0)
    tie_f = jnp.where((rj == rk) & jlt, 1.0, 0.0)
    rank_ref[...] = jnp.sum(lt_f + tie_f, axis=1).astype(jnp.int32)


def _row_copy_body(cost_ref, row_ref, cost_out_ref):
    row_ref[...] = jnp.zeros(row_ref.shape, row_ref.dtype)
    cost_out_ref[...] = cost_ref[...]


def _make_sc_scatter(b, c, embed_dim):
    half = embed_dim // 2
    n_chunks = c // L
    batches_per_w = b // (NC * NS)
    mesh = plsc.VectorSubcoreMesh(
        core_axis_name="c", subcore_axis_name="s",
        num_cores=NC, num_subcores=NS)

    @functools.partial(
        pl.kernel, mesh=mesh,
        out_type=jax.ShapeDtypeStruct((b, c, embed_dim), jnp.float32),
        compiler_params=pltpu.CompilerParams(needs_layout_passes=False),
        cost_estimate=pl.CostEstimate(
            flops=0, transcendentals=0,
            bytes_accessed=b * c * embed_dim * 4),
        scratch_types=[
            pltpu.VMEM((c,), jnp.int32),
            pltpu.VMEM((half, embed_dim), jnp.float32),
            pltpu.VMEM((half, embed_dim), jnp.float32),
        ],
    )
    def sc_scatter(rank_hbm, zpan_hbm, out_hbm, rk_v, pan_a, pan_b):
        wid = lax.axis_index("s") * NC + lax.axis_index("c")
        zero16 = jnp.zeros((L,), jnp.float32)
        one16 = jnp.full((L,), 1.0, jnp.float32)
        kio0 = lax.iota(jnp.int32, L)

        pltpu.sync_copy(zpan_hbm, pan_a)
        pltpu.sync_copy(zpan_hbm, pan_b)

        def scatter_val(val16):
            for ch in range(n_chunks):
                rv = rk_v[pl.ds(ch * L, L)]
                kio = kio0 + (ch * L)
                m_a = rv < half
                plsc.store_scatter(pan_a, [rv, kio], val16, mask=m_a)
                plsc.store_scatter(pan_b, [rv - half, kio], val16,
                                   mask=jnp.logical_not(m_a))

        base = wid * batches_per_w

        def batch_body(i, carry):
            bi = base + i
            pltpu.sync_copy(rank_hbm.at[bi], rk_v)
            scatter_val(one16)
            pltpu.sync_copy(pan_a, out_hbm.at[bi, pl.ds(0, half)])
            pltpu.sync_copy(pan_b, out_hbm.at[bi, pl.ds(half, half)])
            scatter_val(zero16)
            return carry

        lax.fori_loop(0, batches_per_w, batch_body, 0)

    return sc_scatter


def kernel(cost_matrix, rand):
    b, r, c = cost_matrix.shape
    embed_dim = 256
    grid = (b // BB,)
    rank = pl.pallas_call(
        _rank_body,
        grid=grid,
        in_specs=[pl.BlockSpec((BB, c), lambda i: (i, 0))],
        out_specs=pl.BlockSpec((BB, c), lambda i: (i, 0)),
        out_shape=jax.ShapeDtypeStruct((b, c), jnp.int32),
    )(rand)
    zpan = jnp.zeros((embed_dim // 2, embed_dim), jnp.float32)
    col_emb = _make_sc_scatter(b, c, embed_dim)(rank, zpan)
    row_emb, cost_out = pl.pallas_call(
        _row_copy_body,
        grid=grid,
        in_specs=[pl.BlockSpec((BB, r, c), lambda i: (i, 0, 0))],
        out_specs=[
            pl.BlockSpec((BB, r, embed_dim), lambda i: (i, 0, 0)),
            pl.BlockSpec((BB, r, c), lambda i: (i, 0, 0)),
        ],
        out_shape=[
            jax.ShapeDtypeStruct((b, r, embed_dim), cost_matrix.dtype),
            jax.ShapeDtypeStruct((b, r, c), cost_matrix.dtype),
        ],
    )(cost_matrix)
    return (row_emb, col_emb.astype(cost_matrix.dtype), cost_out)


# speedup vs baseline: 1.0278x; 1.0278x over previous
import functools

import jax
import jax.numpy as jnp
from jax import lax
from jax.experimental import pallas as pl
from jax.experimental.pallas import tpu as pltpu
from jax.experimental.pallas import tpu_sc as plsc

BB = 8
NC, NS = 2, 16
L = 16


def _rank_copy_body(rand_ref, cost_ref, rank_ref, cost_out_ref):
    r = rand_ref[...]
    bb, c = r.shape
    rj = r[:, :, None]
    rk = r[:, None, :]
    jlt = jax.lax.broadcasted_iota(jnp.int32, (bb, c, c), 1) < \
        jax.lax.broadcasted_iota(jnp.int32, (bb, c, c), 2)
    lt_f = jnp.where(rj < rk, 1.0, 0.0)
    tie_f = jnp.where((rj == rk) & jlt, 1.0, 0.0)
    rank_ref[...] = jnp.sum(lt_f + tie_f, axis=1).astype(jnp.int32)
    cost_out_ref[...] = cost_ref[...]


def _row_body(row_ref):
    row_ref[...] = jnp.zeros(row_ref.shape, row_ref.dtype)


def _make_sc_scatter(b, c, embed_dim):
    half = embed_dim // 2
    n_chunks = c // L
    batches_per_w = b // (NC * NS)
    mesh = plsc.VectorSubcoreMesh(
        core_axis_name="c", subcore_axis_name="s",
        num_cores=NC, num_subcores=NS)

    @functools.partial(
        pl.kernel, mesh=mesh,
        out_type=jax.ShapeDtypeStruct((b, c, embed_dim), jnp.float32),
        compiler_params=pltpu.CompilerParams(needs_layout_passes=False),
        cost_estimate=pl.CostEstimate(
            flops=0, transcendentals=0,
            bytes_accessed=b * c * embed_dim * 4),
        scratch_types=[
            pltpu.VMEM((c,), jnp.int32),
            pltpu.VMEM((half, embed_dim), jnp.float32),
            pltpu.VMEM((half, embed_dim), jnp.float32),
        ],
    )
    def sc_scatter(rank_hbm, zpan_hbm, out_hbm, rk_v, pan_a, pan_b):
        wid = lax.axis_index("s") * NC + lax.axis_index("c")
        zero16 = jnp.zeros((L,), jnp.float32)
        one16 = jnp.full((L,), 1.0, jnp.float32)
        kio0 = lax.iota(jnp.int32, L)

        pltpu.sync_copy(zpan_hbm, pan_a)
        pltpu.sync_copy(zpan_hbm, pan_b)

        def scatter_val(val16):
            for ch in range(n_chunks):
                rv = rk_v[pl.ds(ch * L, L)]
                kio = kio0 + (ch * L)
                m_a = rv < half
                plsc.store_scatter(pan_a, [rv, kio], val16, mask=m_a)
                plsc.store_scatter(pan_b, [rv - half, kio], val16,
                                   mask=jnp.logical_not(m_a))

        base = wid * batches_per_w

        def batch_body(i, carry):
            bi = base + i
            pltpu.sync_copy(rank_hbm.at[bi], rk_v)
            scatter_val(one16)
            pltpu.sync_copy(pan_a, out_hbm.at[bi, pl.ds(0, half)])
            pltpu.sync_copy(pan_b, out_hbm.at[bi, pl.ds(half, half)])
            scatter_val(zero16)
            return carry

        lax.fori_loop(0, batches_per_w, batch_body, 0)

    return sc_scatter


def kernel(cost_matrix, rand):
    b, r, c = cost_matrix.shape
    embed_dim = 256
    grid = (b // BB,)
    rank, cost_out = pl.pallas_call(
        _rank_copy_body,
        grid=grid,
        in_specs=[
            pl.BlockSpec((BB, c), lambda i: (i, 0)),
            pl.BlockSpec((BB, r, c), lambda i: (i, 0, 0)),
        ],
        out_specs=[
            pl.BlockSpec((BB, c), lambda i: (i, 0)),
            pl.BlockSpec((BB, r, c), lambda i: (i, 0, 0)),
        ],
        out_shape=[
            jax.ShapeDtypeStruct((b, c), jnp.int32),
            jax.ShapeDtypeStruct((b, r, c), cost_matrix.dtype),
        ],
    )(rand, cost_matrix)
    zpan = jnp.zeros((embed_dim // 2, embed_dim), jnp.float32)
    col_emb = _make_sc_scatter(b, c, embed_dim)(rank, zpan)
    row_emb = pl.pallas_call(
        _row_body,
        grid=grid,
        out_specs=pl.BlockSpec((BB, r, embed_dim), lambda i: (i, 0, 0)),
        out_shape=jax.ShapeDtypeStruct((b, r, embed_dim), cost_matrix.dtype),
    )()
    return (row_emb, col_emb.astype(cost_matrix.dtype), cost_out)


# speedup vs baseline: 1.0682x; 1.0393x over previous
import functools

import jax
import jax.numpy as jnp
from jax import lax
from jax.experimental import pallas as pl
from jax.experimental.pallas import tpu as pltpu
from jax.experimental.pallas import tpu_sc as plsc

BB = 8
NC, NS = 2, 16
L = 16


def _rank_copy_body(rand_ref, cost_ref, rank_ref, cost_out_ref):
    r = rand_ref[...]
    bb, c = r.shape
    rj = r[:, :, None]
    rk = r[:, None, :]
    jlt = jax.lax.broadcasted_iota(jnp.int32, (bb, c, c), 1) < \
        jax.lax.broadcasted_iota(jnp.int32, (bb, c, c), 2)
    lt_f = jnp.where(rj < rk, 1.0, 0.0)
    tie_f = jnp.where((rj == rk) & jlt, 1.0, 0.0)
    rank_ref[...] = jnp.sum(lt_f + tie_f, axis=1).astype(jnp.int32)
    cost_out_ref[...] = cost_ref[...]


def _row_body(row_ref):
    row_ref[...] = jnp.zeros(row_ref.shape, row_ref.dtype)


def _make_sc_scatter(b, c, embed_dim):
    half = embed_dim // 2
    n_chunks = c // L
    batches_per_w = b // (NC * NS)
    mesh = plsc.VectorSubcoreMesh(
        core_axis_name="c", subcore_axis_name="s",
        num_cores=NC, num_subcores=NS)

    @functools.partial(
        pl.kernel, mesh=mesh,
        out_type=jax.ShapeDtypeStruct((b, c, embed_dim), jnp.float32),
        compiler_params=pltpu.CompilerParams(needs_layout_passes=False),
        cost_estimate=pl.CostEstimate(
            flops=0, transcendentals=0,
            bytes_accessed=b * c * embed_dim * 4),
        scratch_types=[
            pltpu.VMEM((c,), jnp.int32),
            pltpu.VMEM((c,), jnp.int32),
            pltpu.VMEM((half, embed_dim), jnp.float32),
            pltpu.VMEM((half, embed_dim), jnp.float32),
            pltpu.SemaphoreType.DMA,
            pltpu.SemaphoreType.DMA,
        ],
    )
    def sc_scatter(rank_hbm, zpan_hbm, out_hbm, rk0, rk1, pan_a, pan_b,
                   sem_p, sem_r):
        wid = lax.axis_index("s") * NC + lax.axis_index("c")
        zero16 = jnp.zeros((L,), jnp.float32)
        one16 = jnp.full((L,), 1.0, jnp.float32)
        kio0 = lax.iota(jnp.int32, L)

        pltpu.sync_copy(zpan_hbm, pan_a)
        pltpu.sync_copy(zpan_hbm, pan_b)

        def scatter_val(rk_v, val16):
            for ch in range(n_chunks):
                rv = rk_v[pl.ds(ch * L, L)]
                kio = kio0 + (ch * L)
                m_a = rv < half
                plsc.store_scatter(pan_a, [rv, kio], val16, mask=m_a)
                plsc.store_scatter(pan_b, [rv - half, kio], val16,
                                   mask=jnp.logical_not(m_a))

        base = wid * batches_per_w
        last = base + batches_per_w - 1

        def batch_step(bi, rk_cur, rk_next):
            scatter_val(rk_cur, one16)
            d_a = pltpu.async_copy(pan_a, out_hbm.at[bi, pl.ds(0, half)],
                                   sem_p)
            d_b = pltpu.async_copy(pan_b, out_hbm.at[bi, pl.ds(half, half)],
                                   sem_p)
            d_r = pltpu.async_copy(
                rank_hbm.at[jnp.minimum(bi + 1, last)], rk_next, sem_r)
            d_a.wait()
            d_b.wait()
            scatter_val(rk_cur, zero16)
            d_r.wait()

        pltpu.sync_copy(rank_hbm.at[base], rk0)

        def pair_body(i, carry):
            b0 = base + 2 * i
            batch_step(b0, rk0, rk1)
            batch_step(b0 + 1, rk1, rk0)
            return carry

        lax.fori_loop(0, batches_per_w // 2, pair_body, 0)

    return sc_scatter


def kernel(cost_matrix, rand):
    b, r, c = cost_matrix.shape
    embed_dim = 256
    grid = (b // BB,)
    rank, cost_out = pl.pallas_call(
        _rank_copy_body,
        grid=grid,
        in_specs=[
            pl.BlockSpec((BB, c), lambda i: (i, 0)),
            pl.BlockSpec((BB, r, c), lambda i: (i, 0, 0)),
        ],
        out_specs=[
            pl.BlockSpec((BB, c), lambda i: (i, 0)),
            pl.BlockSpec((BB, r, c), lambda i: (i, 0, 0)),
        ],
        out_shape=[
            jax.ShapeDtypeStruct((b, c), jnp.int32),
            jax.ShapeDtypeStruct((b, r, c), cost_matrix.dtype),
        ],
    )(rand, cost_matrix)
    zpan = jnp.zeros((embed_dim // 2, embed_dim), jnp.float32)
    col_emb = _make_sc_scatter(b, c, embed_dim)(rank, zpan)
    row_emb = pl.pallas_call(
        _row_body,
        grid=grid,
        out_specs=pl.BlockSpec((BB, r, embed_dim), lambda i: (i, 0, 0)),
        out_shape=jax.ShapeDtypeStruct((b, r, embed_dim), cost_matrix.dtype),
    )()
    return (row_emb, col_emb.astype(cost_matrix.dtype), cost_out)
